# baseline (speedup 1.0000x reference)
import functools

import jax
import jax.numpy as jnp
from jax import lax
from jax.experimental import pallas as pl
from jax.experimental.pallas import tpu as pltpu
from jax.experimental.pallas import tpu_sc as plsc

N = 320000
D = 128
S = 10000
NCHUNK = 16
CH = N // NCHUNK
R = 200
NB = CH // R
NG = (R + 15) // 16
HF = D // 2
KV = HF // 16
RING = 4
REC = 6 * HF
IDP = R + 32


def _store_slot(slot_ref, base, sums, cnt):
  for r in range(5):
    for k in range(KV):
      slot_ref[pl.ds(base + r * HF + 16 * k, 16)] = sums[r * KV + k]
  slot_ref[pl.ds(base + 5 * HF, 16)] = jnp.full(
      (16,), cnt.astype(jnp.float32), dtype=jnp.float32)


def _rsqrt16(v):
  i = plsc.bitcast(v, jnp.int32)
  y = plsc.bitcast(jnp.int32(0x5F3759DF) - (i >> 1), jnp.float32)
  hv = -0.5 * v
  for _ in range(3):
    y = y * (1.5 + hv * y * y)
  return y


def _corr_vregs(sums, pv):
  out = []
  for k in range(KV):
    mux = sums[0 * KV + k] * pv
    muy = sums[1 * KV + k] * pv
    cov = sums[4 * KV + k] * pv - mux * muy
    vx = sums[2 * KV + k] * pv - mux * mux
    vy = sums[3 * KV + k] * pv - muy * muy
    out.append(cov * _rsqrt16(vx * vy))
  return out


def _sc_moments(x, y, idx):
  mesh = plsc.VectorSubcoreMesh(core_axis_name="c", subcore_axis_name="s")

  def body(x_hbm, y_hbm, idx_hbm, out_hbm,
           xbuf, ybuf, idbuf, stage, rec_stage, hdr_stage,
           recbuf, hdrbuf, accbuf, fin_stage, shared_rec, shared_hdr,
           in_sem, flush_sem):
    half = lax.axis_index("c")
    chunk = lax.axis_index("s")
    f0 = half * HF
    row0 = chunk * CH

    def in_copies(b, q):
      r0 = row0 + b * R
      return (
          pltpu.make_async_copy(
              x_hbm.at[pl.ds(r0, R), pl.ds(f0, HF)], xbuf.at[q], in_sem),
          pltpu.make_async_copy(
              y_hbm.at[pl.ds(r0, R), pl.ds(f0, HF)], ybuf.at[q], in_sem),
          pltpu.make_async_copy(
              idx_hbm.at[pl.ds(r0, R)],
              idbuf.at[pl.ds(q * IDP + 16, R)], in_sem),
      )

    def issue(b, q):
      for c_ in in_copies(b, q):
        c_.start()

    def wait(b, q):
      for c_ in in_copies(b, q):
        c_.wait()

    def drain_one_flush():
      pltpu.make_async_copy(
          stage.at[pl.ds(0, HF)], out_hbm.at[pl.ds(0, HF)],
          flush_sem).wait()

    issue(0, 0)
    wait(0, 0)
    first_id = idbuf[pl.ds(16, 16)][0]

    zero16 = jnp.zeros((16,), jnp.float32)
    lane = lax.iota(jnp.int32, 16)

    def flush_seg(cur_id, cnt, rp, inflight, sums):
      is_first = cur_id == first_id

      def write_record(rp, inflight):
        _store_slot(rec_stage, 0, sums, cnt)
        pltpu.sync_copy(
            rec_stage, shared_rec.at[pl.ds(chunk * 2 * REC, REC)])
        return rp, inflight

      def write_direct(rp, inflight):
        need = inflight >= RING

        @pl.when(need)
        def _():
          drain_one_flush()

        pv = 1.0 / jnp.full((16,), cnt.astype(jnp.float32),
                            dtype=jnp.float32)
        corr = _corr_vregs(sums, pv)
        for k in range(KV):
          stage[pl.ds(rp * HF + 16 * k, 16)] = corr[k]
        pltpu.async_copy(
            stage.at[pl.ds(rp * HF, HF)],
            out_hbm.at[pl.ds(cur_id * D + f0, HF)], flush_sem)
        return ((rp + 1) & (RING - 1),
                inflight + 1 - need.astype(jnp.int32))

      return lax.cond(is_first, write_record, write_direct, rp, inflight)

    def block_body_p(p, b, carry):
      base = p * IDP + 16

      @pl.when(b > 0)
      def _():
        wait(b, p)

      @pl.when(b + 1 < NB)
      def _():
        issue(b + 1, 1 - p)

      cur_id, cnt, rp, inflight = carry[:4]
      sums = carry[4:]

      idbuf[pl.ds(p * IDP, 16)] = jnp.full((16,), cur_id, dtype=jnp.int32)
      np_ = jnp.int32(0)
      for g in range(NG):
        idv = idbuf[pl.ds(base + g * 16, 16)]
        prevv = idbuf[pl.ds(base + g * 16 - 1, 16)]
        m = idv != prevv
        if (g + 1) * 16 > R:
          m = jnp.logical_and(m, lane < (R - g * 16))
        np_ = np_ + plsc.all_reduce_population_count(m)[0]

      def row_acc(i, sums):
        sums = list(sums)
        for k in range(KV):
          xv = xbuf[p, i, pl.ds(16 * k, 16)]
          yv = ybuf[p, i, pl.ds(16 * k, 16)]
          sums[0 * KV + k] = sums[0 * KV + k] + xv
          sums[1 * KV + k] = sums[1 * KV + k] + yv
          sums[2 * KV + k] = sums[2 * KV + k] + xv * xv
          sums[3 * KV + k] = sums[3 * KV + k] + yv * yv
          sums[4 * KV + k] = sums[4 * KV + k] + xv * yv
        return tuple(sums)

      def load_row(i):
        return tuple(xbuf[p, i, pl.ds(16 * k, 16)] for k in range(KV)) + \
            tuple(ybuf[p, i, pl.ds(16 * k, 16)] for k in range(KV))

      def acc_vals(vals, sums):
        sums = list(sums)
        for k in range(KV):
          xv = vals[k]
          yv = vals[KV + k]
          sums[0 * KV + k] = sums[0 * KV + k] + xv
          sums[1 * KV + k] = sums[1 * KV + k] + yv
          sums[2 * KV + k] = sums[2 * KV + k] + xv * xv
          sums[3 * KV + k] = sums[3 * KV + k] + yv * yv
          sums[4 * KV + k] = sums[4 * KV + k] + xv * yv
        return tuple(sums)

      def run_acc(pos, end, sums):
        def nonempty(sums):
          def step(i, st):
            vals, sums = st[:2 * KV], st[2 * KV:]
            return load_row(i + 1) + acc_vals(vals, sums)

          st = load_row(pos) + tuple(sums)
          st = lax.fori_loop(pos, end - 1, step, st)
          return acc_vals(st[:2 * KV], st[2 * KV:])

        def empty(sums):
          return tuple(sums)

        return lax.cond(pos < end, nonempty, empty, tuple(sums))

      def find_end(pos, vid):
        vidv = jnp.full((16,), vid, dtype=jnp.int32)

        def fstep(g, found):
          def scan():
            q0 = g * 16
            w = idbuf[pl.ds(base + q0, 16)]
            m = jnp.logical_and(w != vidv, q0 + lane >= pos)
            f = plsc.all_reduce_ffs(m)[0]
            return jnp.where(f < 16, q0 + f, jnp.int32(-1))

          return lax.cond(found < 0, scan, lambda: found)

        found = lax.fori_loop(pos >> 4, NG, fstep, jnp.int32(-1))
        return found

      def seg_step(j, scarry):
        (pos, cur_id, cnt, rp, inflight) = scarry[:5]
        sums = scarry[5:]
        end = find_end(pos, cur_id)
        sums = run_acc(pos, end, tuple(sums))
        cnt = cnt + (end - pos)
        rp, inflight = flush_seg(cur_id, cnt, rp, inflight, sums)
        nid = idbuf[pl.ds(base + end, 16)][0]
        return (end, nid, jnp.int32(0), rp, inflight) + (zero16,) * 20

      scarry = (jnp.int32(0),) + tuple(carry)
      scarry = lax.fori_loop(0, np_, seg_step, scarry)
      (start, cur_id, cnt, rp, inflight) = scarry[:5]
      sums = run_acc(start, jnp.int32(R), tuple(scarry[5:]))
      cnt = cnt + (R - start)
      return (cur_id, cnt, rp, inflight) + tuple(sums)

    def block_pair(b2, carry):
      for p_ in (0, 1):
        carry = block_body_p(p_, b2 * 2 + p_, carry)
      return carry

    carry0 = (first_id, jnp.int32(0), jnp.int32(0), jnp.int32(0)) + \
        (zero16,) * 20
    carry = lax.fori_loop(0, NB // 2, block_pair, carry0)

    cur_id, cnt = carry[0], carry[1]
    inflight = carry[3]
    sums = carry[4:]

    last_diff = cur_id != first_id
    _store_slot(rec_stage, 0, sums, cnt)
    slot = jnp.where(last_diff, 1, 0).astype(jnp.int32)
    pltpu.sync_copy(
        rec_stage, shared_rec.at[pl.ds((chunk * 2 + slot) * REC, REC)])
    hdr_stage[pl.ds(0, 16)] = jnp.full((16,), first_id, dtype=jnp.int32)
    hdr_stage[pl.ds(16, 16)] = jnp.full(
        (16,), jnp.where(last_diff, cur_id, -1), dtype=jnp.int32)
    pltpu.sync_copy(hdr_stage, shared_hdr.at[pl.ds(chunk * 32, 32)])

    for j in range(RING):
      @pl.when(j < inflight)
      def _():
        drain_one_flush()

    plsc.subcore_barrier()

    @pl.when(chunk == 0)
    def _merge():
      pltpu.sync_copy(shared_rec, recbuf)
      pltpu.sync_copy(shared_hdr, hdrbuf)

      def flush_acc(prev):
        cntv = accbuf[pl.ds(5 * HF, 16)]
        pv = 1.0 / cntv
        moms = [accbuf[pl.ds(r * HF + 16 * k, 16)]
                for r in range(5) for k in range(KV)]
        corr = _corr_vregs(moms, pv)
        for k in range(KV):
          fin_stage[pl.ds(16 * k, 16)] = corr[k]
        pltpu.sync_copy(fin_stage, out_hbm.at[pl.ds(prev * D + f0, HF)])

      def slot_step(t, prev):
        hid = hdrbuf[pl.ds(t * 16, 16)][0]
        rb = t * REC

        def live(prev):
          same = hid == prev

          def addacc():
            for k in range(6 * KV):
              accbuf[pl.ds(16 * k, 16)] = (
                  accbuf[pl.ds(16 * k, 16)]
                  + recbuf[pl.ds(rb + 16 * k, 16)])

          def newacc():
            @pl.when(prev >= 0)
            def _():
              flush_acc(prev)
            for k in range(6 * KV):
              accbuf[pl.ds(16 * k, 16)] = recbuf[pl.ds(rb + 16 * k, 16)]

          lax.cond(same, addacc, newacc)
          return hid

        return lax.cond(hid >= 0, live, lambda p_: p_, prev)

      prev = lax.fori_loop(0, NCHUNK * 2, slot_step, jnp.int32(-1))

      @pl.when(prev >= 0)
      def _():
        flush_acc(prev)

  return pl.kernel(
      body,
      out_type=jax.ShapeDtypeStruct((S * D,), jnp.float32),
      mesh=mesh,
      compiler_params=pltpu.CompilerParams(use_tc_tiling_on_sc=False,
                                           needs_layout_passes=False),
      scratch_types=[
          pltpu.VMEM((2, R, HF), jnp.float32),
          pltpu.VMEM((2, R, HF), jnp.float32),
          pltpu.VMEM((2 * IDP,), jnp.int32),
          pltpu.VMEM((RING * HF,), jnp.float32),
          pltpu.VMEM((REC,), jnp.float32),
          pltpu.VMEM((32,), jnp.int32),
          pltpu.VMEM((NCHUNK * 2 * REC,), jnp.float32),
          pltpu.VMEM((NCHUNK * 2 * 16,), jnp.int32),
          pltpu.VMEM((REC,), jnp.float32),
          pltpu.VMEM((HF,), jnp.float32),
          pltpu.VMEM_SHARED((NCHUNK * 2 * REC,), jnp.float32),
          pltpu.VMEM_SHARED((NCHUNK * 2 * 16,), jnp.int32),
          pltpu.SemaphoreType.DMA,
          pltpu.SemaphoreType.DMA,
      ],
  )(x, y, idx)


def kernel(input, target, batch_idx):
  x = input.reshape(N, D).astype(jnp.float32)
  y = target.reshape(N, D).astype(jnp.float32)
  idx = batch_idx.reshape(N).astype(jnp.int32)
  return _sc_moments(x, y, idx).reshape(S, D)


# speedup vs baseline: 1.2858x; 1.2858x over previous
import functools

import jax
import jax.numpy as jnp
from jax import lax
from jax.experimental import pallas as pl
from jax.experimental.pallas import tpu as pltpu
from jax.experimental.pallas import tpu_sc as plsc

N = 320000
D = 128
S = 10000
NCHUNK = 16
CH = N // NCHUNK
R = 200
NB = CH // R
NG = (R + 15) // 16
HF = D // 2
KV = HF // 16
RING = 4
REC = 6 * HF
IDP = R + 32


def _store_slot(slot_ref, base, sums, cnt):
  for r in range(5):
    for k in range(KV):
      slot_ref[pl.ds(base + r * HF + 16 * k, 16)] = sums[r * KV + k]
  slot_ref[pl.ds(base + 5 * HF, 16)] = jnp.full(
      (16,), cnt.astype(jnp.float32), dtype=jnp.float32)


def _rsqrt16(v):
  i = plsc.bitcast(v, jnp.int32)
  y = plsc.bitcast(jnp.int32(0x5F3759DF) - (i >> 1), jnp.float32)
  hv = -0.5 * v
  for _ in range(3):
    y = y * (1.5 + hv * y * y)
  return y


def _corr_vregs(sums, pv):
  out = []
  for k in range(KV):
    mux = sums[0 * KV + k] * pv
    muy = sums[1 * KV + k] * pv
    cov = sums[4 * KV + k] * pv - mux * muy
    vx = sums[2 * KV + k] * pv - mux * mux
    vy = sums[3 * KV + k] * pv - muy * muy
    out.append(cov * _rsqrt16(vx * vy))
  return out


def _sc_moments(x, y, idx):
  mesh = plsc.VectorSubcoreMesh(core_axis_name="c", subcore_axis_name="s")

  def body(x_hbm, y_hbm, idx_hbm, out_hbm,
           xbuf, ybuf, idbuf, stage, rec_stage, hdr_stage,
           recbuf, hdrbuf, accbuf, fin_stage, shared_rec, shared_hdr,
           in_sem, flush_sem):
    half = lax.axis_index("c")
    chunk = lax.axis_index("s")
    f0 = half * HF
    row0 = chunk * CH

    def in_copies(b, q):
      r0 = row0 + b * R
      return (
          pltpu.make_async_copy(
              x_hbm.at[pl.ds(r0, R), pl.ds(f0, HF)], xbuf.at[q], in_sem),
          pltpu.make_async_copy(
              y_hbm.at[pl.ds(r0, R), pl.ds(f0, HF)], ybuf.at[q], in_sem),
          pltpu.make_async_copy(
              idx_hbm.at[pl.ds(r0, R)],
              idbuf.at[pl.ds(q * IDP + 16, R)], in_sem),
      )

    def issue(b, q):
      for c_ in in_copies(b, q):
        c_.start()

    def wait(b, q):
      for c_ in in_copies(b, q):
        c_.wait()

    def drain_one_flush():
      pltpu.make_async_copy(
          stage.at[pl.ds(0, HF)], out_hbm.at[pl.ds(0, HF)],
          flush_sem).wait()

    issue(0, 0)
    wait(0, 0)
    first_id = idbuf[pl.ds(16, 16)][0]

    zero16 = jnp.zeros((16,), jnp.float32)
    lane = lax.iota(jnp.int32, 16)

    def flush_seg(cur_id, cnt, rp, inflight, sums):
      is_first = cur_id == first_id

      def write_record(rp, inflight):
        _store_slot(rec_stage, 0, sums, cnt)
        pltpu.sync_copy(
            rec_stage, shared_rec.at[pl.ds(chunk * 2 * REC, REC)])
        return rp, inflight

      def write_direct(rp, inflight):
        need = inflight >= RING

        @pl.when(need)
        def _():
          drain_one_flush()

        pv = 1.0 / jnp.full((16,), cnt.astype(jnp.float32),
                            dtype=jnp.float32)
        corr = _corr_vregs(sums, pv)
        for k in range(KV):
          stage[pl.ds(rp * HF + 16 * k, 16)] = corr[k]
        pltpu.async_copy(
            stage.at[pl.ds(rp * HF, HF)],
            out_hbm.at[pl.ds(cur_id * D + f0, HF)], flush_sem)
        return ((rp + 1) & (RING - 1),
                inflight + 1 - need.astype(jnp.int32))

      return lax.cond(is_first, write_record, write_direct, rp, inflight)

    def block_body_p(p, b, carry):
      base = p * IDP + 16

      @pl.when(b > 0)
      def _():
        wait(b, p)

      @pl.when(b + 1 < NB)
      def _():
        issue(b + 1, 1 - p)

      cur_id, cnt, rp, inflight = carry[:4]
      sums = carry[4:]

      idbuf[pl.ds(p * IDP, 16)] = jnp.full((16,), cur_id, dtype=jnp.int32)
      np_ = jnp.int32(0)
      for g in range(NG):
        idv = idbuf[pl.ds(base + g * 16, 16)]
        prevv = idbuf[pl.ds(base + g * 16 - 1, 16)]
        m = idv != prevv
        if (g + 1) * 16 > R:
          m = jnp.logical_and(m, lane < (R - g * 16))
        np_ = np_ + plsc.all_reduce_population_count(m)[0]

      def row_acc(i, sums):
        sums = list(sums)
        for k in range(KV):
          xv = xbuf[p, i, pl.ds(16 * k, 16)]
          yv = ybuf[p, i, pl.ds(16 * k, 16)]
          sums[0 * KV + k] = sums[0 * KV + k] + xv
          sums[1 * KV + k] = sums[1 * KV + k] + yv
          sums[2 * KV + k] = sums[2 * KV + k] + xv * xv
          sums[3 * KV + k] = sums[3 * KV + k] + yv * yv
          sums[4 * KV + k] = sums[4 * KV + k] + xv * yv
        return tuple(sums)

      def run_acc(pos, end, sums):
        n = end - pos

        def quad(j, sums):
          i = pos + 4 * j
          xs = [xbuf[p, i + u, pl.ds(16 * k, 16)]
                for u in range(4) for k in range(KV)]
          ys = [ybuf[p, i + u, pl.ds(16 * k, 16)]
                for u in range(4) for k in range(KV)]
          sums = list(sums)
          for u in range(4):
            for k in range(KV):
              xv = xs[u * KV + k]
              yv = ys[u * KV + k]
              sums[0 * KV + k] = sums[0 * KV + k] + xv
              sums[1 * KV + k] = sums[1 * KV + k] + yv
              sums[2 * KV + k] = sums[2 * KV + k] + xv * xv
              sums[3 * KV + k] = sums[3 * KV + k] + yv * yv
              sums[4 * KV + k] = sums[4 * KV + k] + xv * yv
          return tuple(sums)

        sums = lax.fori_loop(0, n >> 2, quad, tuple(sums))
        tail = pos + (n & ~3)

        def single(j, sums):
          return row_acc(tail + j, sums)

        return lax.fori_loop(0, n & 3, single, sums)

      def find_end(pos, vid):
        vidv = jnp.full((16,), vid, dtype=jnp.int32)

        def fstep(g, found):
          def scan():
            q0 = g * 16
            w = idbuf[pl.ds(base + q0, 16)]
            m = jnp.logical_and(w != vidv, q0 + lane >= pos)
            f = plsc.all_reduce_ffs(m)[0]
            return jnp.where(f < 16, q0 + f, jnp.int32(-1))

          return lax.cond(found < 0, scan, lambda: found)

        found = lax.fori_loop(pos >> 4, NG, fstep, jnp.int32(-1))
        return found

      def seg_step(j, scarry):
        (pos, cur_id, cnt, rp, inflight) = scarry[:5]
        sums = scarry[5:]
        end = find_end(pos, cur_id)
        sums = run_acc(pos, end, tuple(sums))
        cnt = cnt + (end - pos)
        rp, inflight = flush_seg(cur_id, cnt, rp, inflight, sums)
        nid = idbuf[pl.ds(base + end, 16)][0]
        return (end, nid, jnp.int32(0), rp, inflight) + (zero16,) * 20

      scarry = (jnp.int32(0),) + tuple(carry)
      scarry = lax.fori_loop(0, np_, seg_step, scarry)
      (start, cur_id, cnt, rp, inflight) = scarry[:5]
      sums = run_acc(start, jnp.int32(R), tuple(scarry[5:]))
      cnt = cnt + (R - start)
      return (cur_id, cnt, rp, inflight) + tuple(sums)

    def block_pair(b2, carry):
      for p_ in (0, 1):
        carry = block_body_p(p_, b2 * 2 + p_, carry)
      return carry

    carry0 = (first_id, jnp.int32(0), jnp.int32(0), jnp.int32(0)) + \
        (zero16,) * 20
    carry = lax.fori_loop(0, NB // 2, block_pair, carry0)

    cur_id, cnt = carry[0], carry[1]
    inflight = carry[3]
    sums = carry[4:]

    last_diff = cur_id != first_id
    _store_slot(rec_stage, 0, sums, cnt)
    slot = jnp.where(last_diff, 1, 0).astype(jnp.int32)
    pltpu.sync_copy(
        rec_stage, shared_rec.at[pl.ds((chunk * 2 + slot) * REC, REC)])
    hdr_stage[pl.ds(0, 16)] = jnp.full((16,), first_id, dtype=jnp.int32)
    hdr_stage[pl.ds(16, 16)] = jnp.full(
        (16,), jnp.where(last_diff, cur_id, -1), dtype=jnp.int32)
    pltpu.sync_copy(hdr_stage, shared_hdr.at[pl.ds(chunk * 32, 32)])

    for j in range(RING):
      @pl.when(j < inflight)
      def _():
        drain_one_flush()

    plsc.subcore_barrier()

    @pl.when(chunk == 0)
    def _merge():
      pltpu.sync_copy(shared_rec, recbuf)
      pltpu.sync_copy(shared_hdr, hdrbuf)

      def flush_acc(prev):
        cntv = accbuf[pl.ds(5 * HF, 16)]
        pv = 1.0 / cntv
        moms = [accbuf[pl.ds(r * HF + 16 * k, 16)]
                for r in range(5) for k in range(KV)]
        corr = _corr_vregs(moms, pv)
        for k in range(KV):
          fin_stage[pl.ds(16 * k, 16)] = corr[k]
        pltpu.sync_copy(fin_stage, out_hbm.at[pl.ds(prev * D + f0, HF)])

      def slot_step(t, prev):
        hid = hdrbuf[pl.ds(t * 16, 16)][0]
        rb = t * REC

        def live(prev):
          same = hid == prev

          def addacc():
            for k in range(6 * KV):
              accbuf[pl.ds(16 * k, 16)] = (
                  accbuf[pl.ds(16 * k, 16)]
                  + recbuf[pl.ds(rb + 16 * k, 16)])

          def newacc():
            @pl.when(prev >= 0)
            def _():
              flush_acc(prev)
            for k in range(6 * KV):
              accbuf[pl.ds(16 * k, 16)] = recbuf[pl.ds(rb + 16 * k, 16)]

          lax.cond(same, addacc, newacc)
          return hid

        return lax.cond(hid >= 0, live, lambda p_: p_, prev)

      prev = lax.fori_loop(0, NCHUNK * 2, slot_step, jnp.int32(-1))

      @pl.when(prev >= 0)
      def _():
        flush_acc(prev)

  return pl.kernel(
      body,
      out_type=jax.ShapeDtypeStruct((S * D,), jnp.float32),
      mesh=mesh,
      compiler_params=pltpu.CompilerParams(use_tc_tiling_on_sc=False,
                                           needs_layout_passes=False),
      scratch_types=[
          pltpu.VMEM((2, R, HF), jnp.float32),
          pltpu.VMEM((2, R, HF), jnp.float32),
          pltpu.VMEM((2 * IDP,), jnp.int32),
          pltpu.VMEM((RING * HF,), jnp.float32),
          pltpu.VMEM((REC,), jnp.float32),
          pltpu.VMEM((32,), jnp.int32),
          pltpu.VMEM((NCHUNK * 2 * REC,), jnp.float32),
          pltpu.VMEM((NCHUNK * 2 * 16,), jnp.int32),
          pltpu.VMEM((REC,), jnp.float32),
          pltpu.VMEM((HF,), jnp.float32),
          pltpu.VMEM_SHARED((NCHUNK * 2 * REC,), jnp.float32),
          pltpu.VMEM_SHARED((NCHUNK * 2 * 16,), jnp.int32),
          pltpu.SemaphoreType.DMA,
          pltpu.SemaphoreType.DMA,
      ],
  )(x, y, idx)


def kernel(input, target, batch_idx):
  x = input.reshape(N, D).astype(jnp.float32)
  y = target.reshape(N, D).astype(jnp.float32)
  idx = batch_idx.reshape(N).astype(jnp.int32)
  return _sc_moments(x, y, idx).reshape(S, D)


# speedup vs baseline: 1.3128x; 1.0210x over previous
import functools

import jax
import jax.numpy as jnp
from jax import lax
from jax.experimental import pallas as pl
from jax.experimental.pallas import tpu as pltpu
from jax.experimental.pallas import tpu_sc as plsc

N = 320000
D = 128
S = 10000
NCHUNK = 16
CH = N // NCHUNK
R = 200
NB = CH // R
NG = (R + 15) // 16
HF = D // 2
KV = HF // 16
RING = 8
REC = 6 * HF


def _store_slot(slot_ref, base, sums, cnt):
  for r in range(5):
    for k in range(KV):
      slot_ref[pl.ds(base + r * HF + 16 * k, 16)] = sums[r * KV + k]
  slot_ref[pl.ds(base + 5 * HF, 16)] = jnp.full(
      (16,), cnt.astype(jnp.float32), dtype=jnp.float32)


def _rsqrt16(v):
  i = plsc.bitcast(v, jnp.int32)
  y = plsc.bitcast(jnp.int32(0x5F3759DF) - (i >> 1), jnp.float32)
  hv = -0.5 * v
  for _ in range(3):
    y = y * (1.5 + hv * y * y)
  return y


def _corr_vregs(sums, pv):
  out = []
  for k in range(KV):
    mux = sums[0 * KV + k] * pv
    muy = sums[1 * KV + k] * pv
    cov = sums[4 * KV + k] * pv - mux * muy
    vx = sums[2 * KV + k] * pv - mux * mux
    vy = sums[3 * KV + k] * pv - muy * muy
    out.append(cov * _rsqrt16(vx * vy))
  return out


def _sc_moments(x, y, idx):
  mesh = plsc.VectorSubcoreMesh(core_axis_name="c", subcore_axis_name="s")

  def body(x_hbm, y_hbm, idx_hbm, out_hbm,
           xbuf, ybuf, idbuf, stage, rec_stage, hdr_stage,
           recbuf, hdrbuf, accbuf, fin_stage, shared_rec, shared_hdr,
           in_sem, flush_sem):
    half = lax.axis_index("c")
    chunk = lax.axis_index("s")
    f0 = half * HF
    row0 = chunk * CH

    def in_copies(b, q):
      r0 = row0 + b * R
      return (
          pltpu.make_async_copy(
              x_hbm.at[pl.ds(r0, R), pl.ds(f0, HF)], xbuf.at[q], in_sem),
          pltpu.make_async_copy(
              y_hbm.at[pl.ds(r0, R), pl.ds(f0, HF)], ybuf.at[q], in_sem),
      )

    def issue(b, q):
      for c_ in in_copies(b, q):
        c_.start()

    def wait(b, q):
      for c_ in in_copies(b, q):
        c_.wait()

    def drain_one_flush():
      pltpu.make_async_copy(
          stage.at[pl.ds(0, HF)], out_hbm.at[pl.ds(0, HF)],
          flush_sem).wait()

    issue(0, 0)
    pltpu.sync_copy(idx_hbm.at[pl.ds(row0, CH)], idbuf.at[pl.ds(16, CH)])
    first_id = idbuf[pl.ds(16, 16)][0]
    idbuf[pl.ds(0, 16)] = jnp.full((16,), first_id, dtype=jnp.int32)
    wait(0, 0)

    zero16 = jnp.zeros((16,), jnp.float32)
    lane = lax.iota(jnp.int32, 16)

    def flush_seg(cur_id, cnt, rp, inflight, sums):
      is_first = cur_id == first_id

      def write_record(rp, inflight):
        _store_slot(rec_stage, 0, sums, cnt)
        pltpu.sync_copy(
            rec_stage, shared_rec.at[pl.ds(chunk * 2 * REC, REC)])
        return rp, inflight

      def write_direct(rp, inflight):
        need = inflight >= RING

        @pl.when(need)
        def _():
          drain_one_flush()

        pv = 1.0 / jnp.full((16,), cnt.astype(jnp.float32),
                            dtype=jnp.float32)
        corr = _corr_vregs(sums, pv)
        for k in range(KV):
          stage[pl.ds(rp * HF + 16 * k, 16)] = corr[k]
        pltpu.async_copy(
            stage.at[pl.ds(rp * HF, HF)],
            out_hbm.at[pl.ds(cur_id * D + f0, HF)], flush_sem)
        return ((rp + 1) & (RING - 1),
                inflight + 1 - need.astype(jnp.int32))

      return lax.cond(is_first, write_record, write_direct, rp, inflight)

    def block_body_p(p, b, carry):
      base = 16 + b * R

      @pl.when(b > 0)
      def _():
        wait(b, p)

      @pl.when(b + 1 < NB)
      def _():
        issue(b + 1, 1 - p)

      cur_id, cnt, rp, inflight = carry[:4]
      sums = carry[4:]

      np_ = jnp.int32(0)
      for g in range(NG):
        idv = idbuf[pl.ds(base + g * 16, 16)]
        prevv = idbuf[pl.ds(base + g * 16 - 1, 16)]
        m = idv != prevv
        if (g + 1) * 16 > R:
          m = jnp.logical_and(m, lane < (R - g * 16))
        np_ = np_ + plsc.all_reduce_population_count(m)[0]

      def row_acc(i, sums):
        sums = list(sums)
        for k in range(KV):
          xv = xbuf[p, i, pl.ds(16 * k, 16)]
          yv = ybuf[p, i, pl.ds(16 * k, 16)]
          sums[0 * KV + k] = sums[0 * KV + k] + xv
          sums[1 * KV + k] = sums[1 * KV + k] + yv
          sums[2 * KV + k] = sums[2 * KV + k] + xv * xv
          sums[3 * KV + k] = sums[3 * KV + k] + yv * yv
          sums[4 * KV + k] = sums[4 * KV + k] + xv * yv
        return tuple(sums)

      def run_acc(pos, end, sums):
        return lax.fori_loop(pos, end, row_acc, tuple(sums))

      def find_end(pos, vid):
        vidv = jnp.full((16,), vid, dtype=jnp.int32)

        def fstep(g, found):
          def scan():
            q0 = g * 16
            w = idbuf[pl.ds(base + q0, 16)]
            m = jnp.logical_and(w != vidv, q0 + lane >= pos)
            f = plsc.all_reduce_ffs(m)[0]
            return jnp.where(f < 16, q0 + f, jnp.int32(-1))

          return lax.cond(found < 0, scan, lambda: found)

        found = lax.fori_loop(pos >> 4, NG, fstep, jnp.int32(-1))
        return found

      def seg_step(j, scarry):
        (pos, cur_id, cnt, rp, inflight) = scarry[:5]
        sums = scarry[5:]
        end = find_end(pos, cur_id)
        sums = run_acc(pos, end, tuple(sums))
        cnt = cnt + (end - pos)
        rp, inflight = flush_seg(cur_id, cnt, rp, inflight, sums)
        nid = idbuf[pl.ds(base + end, 16)][0]
        return (end, nid, jnp.int32(0), rp, inflight) + (zero16,) * 20

      scarry = (jnp.int32(0),) + tuple(carry)
      scarry = lax.fori_loop(0, np_, seg_step, scarry)
      (start, cur_id, cnt, rp, inflight) = scarry[:5]
      sums = run_acc(start, jnp.int32(R), tuple(scarry[5:]))
      cnt = cnt + (R - start)
      return (cur_id, cnt, rp, inflight) + tuple(sums)

    def block_pair(b2, carry):
      for p_ in (0, 1):
        carry = block_body_p(p_, b2 * 2 + p_, carry)
      return carry

    carry0 = (first_id, jnp.int32(0), jnp.int32(0), jnp.int32(0)) + \
        (zero16,) * 20
    carry = lax.fori_loop(0, NB // 2, block_pair, carry0)

    cur_id, cnt = carry[0], carry[1]
    inflight = carry[3]
    sums = carry[4:]

    last_diff = cur_id != first_id
    _store_slot(rec_stage, 0, sums, cnt)
    slot = jnp.where(last_diff, 1, 0).astype(jnp.int32)
    pltpu.sync_copy(
        rec_stage, shared_rec.at[pl.ds((chunk * 2 + slot) * REC, REC)])
    hdr_stage[pl.ds(0, 16)] = jnp.full((16,), first_id, dtype=jnp.int32)
    hdr_stage[pl.ds(16, 16)] = jnp.full(
        (16,), jnp.where(last_diff, cur_id, -1), dtype=jnp.int32)
    pltpu.sync_copy(hdr_stage, shared_hdr.at[pl.ds(chunk * 32, 32)])

    for j in range(RING):
      @pl.when(j < inflight)
      def _():
        drain_one_flush()

    plsc.subcore_barrier()

    @pl.when(chunk == 0)
    def _merge():
      pltpu.sync_copy(shared_rec, recbuf)
      pltpu.sync_copy(shared_hdr, hdrbuf)

      def flush_acc(prev):
        cntv = accbuf[pl.ds(5 * HF, 16)]
        pv = 1.0 / cntv
        moms = [accbuf[pl.ds(r * HF + 16 * k, 16)]
                for r in range(5) for k in range(KV)]
        corr = _corr_vregs(moms, pv)
        for k in range(KV):
          fin_stage[pl.ds(16 * k, 16)] = corr[k]
        pltpu.sync_copy(fin_stage, out_hbm.at[pl.ds(prev * D + f0, HF)])

      def slot_step(t, prev):
        hid = hdrbuf[pl.ds(t * 16, 16)][0]
        rb = t * REC

        def live(prev):
          same = hid == prev

          def addacc():
            for k in range(6 * KV):
              accbuf[pl.ds(16 * k, 16)] = (
                  accbuf[pl.ds(16 * k, 16)]
                  + recbuf[pl.ds(rb + 16 * k, 16)])

          def newacc():
            @pl.when(prev >= 0)
            def _():
              flush_acc(prev)
            for k in range(6 * KV):
              accbuf[pl.ds(16 * k, 16)] = recbuf[pl.ds(rb + 16 * k, 16)]

          lax.cond(same, addacc, newacc)
          return hid

        return lax.cond(hid >= 0, live, lambda p_: p_, prev)

      prev = lax.fori_loop(0, NCHUNK * 2, slot_step, jnp.int32(-1))

      @pl.when(prev >= 0)
      def _():
        flush_acc(prev)

  return pl.kernel(
      body,
      out_type=jax.ShapeDtypeStruct((S * D,), jnp.float32),
      mesh=mesh,
      compiler_params=pltpu.CompilerParams(use_tc_tiling_on_sc=False,
                                           needs_layout_passes=False),
      scratch_types=[
          pltpu.VMEM((2, R, HF), jnp.float32),
          pltpu.VMEM((2, R, HF), jnp.float32),
          pltpu.VMEM((CH + 32,), jnp.int32),
          pltpu.VMEM((RING * HF,), jnp.float32),
          pltpu.VMEM((REC,), jnp.float32),
          pltpu.VMEM((32,), jnp.int32),
          pltpu.VMEM((NCHUNK * 2 * REC,), jnp.float32),
          pltpu.VMEM((NCHUNK * 2 * 16,), jnp.int32),
          pltpu.VMEM((REC,), jnp.float32),
          pltpu.VMEM((HF,), jnp.float32),
          pltpu.VMEM_SHARED((NCHUNK * 2 * REC,), jnp.float32),
          pltpu.VMEM_SHARED((NCHUNK * 2 * 16,), jnp.int32),
          pltpu.SemaphoreType.DMA,
          pltpu.SemaphoreType.DMA,
      ],
  )(x, y, idx)


def kernel(input, target, batch_idx):
  x = input.reshape(N, D).astype(jnp.float32)
  y = target.reshape(N, D).astype(jnp.float32)
  idx = batch_idx.reshape(N).astype(jnp.int32)
  return _sc_moments(x, y, idx).reshape(S, D)


# speedup vs baseline: 1.3222x; 1.0072x over previous
import functools

import jax
import jax.numpy as jnp
from jax import lax
from jax.experimental import pallas as pl
from jax.experimental.pallas import tpu as pltpu
from jax.experimental.pallas import tpu_sc as plsc

N = 320000
D = 128
S = 10000
NCHUNK = 16
CH = N // NCHUNK
R = 200
NB = CH // R
NG = (R + 15) // 16
HF = D // 2
KV = HF // 16
RING = 4
REC = 6 * HF
IDP = R + 32


def _store_slot(slot_ref, base, sums, cnt):
  for r in range(5):
    for k in range(KV):
      slot_ref[pl.ds(base + r * HF + 16 * k, 16)] = sums[r * KV + k]
  slot_ref[pl.ds(base + 5 * HF, 16)] = jnp.full(
      (16,), cnt.astype(jnp.float32), dtype=jnp.float32)


def _rsqrt16(v):
  i = plsc.bitcast(v, jnp.int32)
  y = plsc.bitcast(jnp.int32(0x5F3759DF) - (i >> 1), jnp.float32)
  hv = -0.5 * v
  for _ in range(3):
    y = y * (1.5 + hv * y * y)
  return y


def _corr_vregs(sums, pv):
  out = []
  for k in range(KV):
    mux = sums[0 * KV + k] * pv
    muy = sums[1 * KV + k] * pv
    cov = sums[4 * KV + k] * pv - mux * muy
    vx = sums[2 * KV + k] * pv - mux * mux
    vy = sums[3 * KV + k] * pv - muy * muy
    out.append(cov * _rsqrt16(vx * vy))
  return out


def _sc_moments(x, y, idx):
  mesh = plsc.VectorSubcoreMesh(core_axis_name="c", subcore_axis_name="s")

  def body(x_hbm, y_hbm, idx_hbm, out_hbm,
           xbuf, ybuf, idbuf, stage, rec_stage, hdr_stage,
           recbuf, hdrbuf, accbuf, fin_stage, shared_rec, shared_hdr,
           in_sem, flush_sem):
    half = lax.axis_index("c")
    chunk = lax.axis_index("s")
    f0 = half * HF
    row0 = chunk * CH

    def in_copies(b, q):
      r0 = row0 + b * R
      return (
          pltpu.make_async_copy(
              x_hbm.at[pl.ds(r0, R), pl.ds(f0, HF)], xbuf.at[q], in_sem),
          pltpu.make_async_copy(
              y_hbm.at[pl.ds(r0, R), pl.ds(f0, HF)], ybuf.at[q], in_sem),
          pltpu.make_async_copy(
              idx_hbm.at[pl.ds(r0, R)],
              idbuf.at[pl.ds(q * IDP + 16, R)], in_sem),
      )

    def issue(b, q):
      for c_ in in_copies(b, q):
        c_.start()

    def wait(b, q):
      for c_ in in_copies(b, q):
        c_.wait()

    def drain_one_flush():
      pltpu.make_async_copy(
          stage.at[pl.ds(0, HF)], out_hbm.at[pl.ds(0, HF)],
          flush_sem).wait()

    issue(0, 0)
    wait(0, 0)
    first_id = idbuf[pl.ds(16, 16)][0]

    zero16 = jnp.zeros((16,), jnp.float32)
    lane = lax.iota(jnp.int32, 16)

    def flush_seg(cur_id, cnt, rp, inflight, sums):
      is_first = cur_id == first_id

      def write_record(rp, inflight):
        _store_slot(rec_stage, 0, sums, cnt)
        pltpu.sync_copy(
            rec_stage, shared_rec.at[pl.ds(chunk * 2 * REC, REC)])
        return rp, inflight

      def write_direct(rp, inflight):
        need = inflight >= RING

        @pl.when(need)
        def _():
          drain_one_flush()

        pv = 1.0 / jnp.full((16,), cnt.astype(jnp.float32),
                            dtype=jnp.float32)
        corr = _corr_vregs(sums, pv)
        for k in range(KV):
          stage[pl.ds(rp * HF + 16 * k, 16)] = corr[k]
        pltpu.async_copy(
            stage.at[pl.ds(rp * HF, HF)],
            out_hbm.at[pl.ds(cur_id * D + f0, HF)], flush_sem)
        return ((rp + 1) & (RING - 1),
                inflight + 1 - need.astype(jnp.int32))

      return lax.cond(is_first, write_record, write_direct, rp, inflight)

    def block_body_p(p, b, carry):
      base = p * IDP + 16

      @pl.when(b > 0)
      def _():
        wait(b, p)

      @pl.when(b + 1 < NB)
      def _():
        issue(b + 1, 1 - p)

      cur_id, cnt, rp, inflight = carry[:4]
      sums = carry[4:]

      idbuf[pl.ds(p * IDP, 16)] = jnp.full((16,), cur_id, dtype=jnp.int32)
      np_ = jnp.int32(0)
      for g in range(NG):
        idv = idbuf[pl.ds(base + g * 16, 16)]
        prevv = idbuf[pl.ds(base + g * 16 - 1, 16)]
        m = idv != prevv
        if (g + 1) * 16 > R:
          m = jnp.logical_and(m, lane < (R - g * 16))
        np_ = np_ + plsc.all_reduce_population_count(m)[0]

      def row_acc(i, sums):
        sums = list(sums)
        for k in range(KV):
          xv = xbuf[p, i, pl.ds(16 * k, 16)]
          yv = ybuf[p, i, pl.ds(16 * k, 16)]
          sums[0 * KV + k] = sums[0 * KV + k] + xv
          sums[1 * KV + k] = sums[1 * KV + k] + yv
          sums[2 * KV + k] = sums[2 * KV + k] + xv * xv
          sums[3 * KV + k] = sums[3 * KV + k] + yv * yv
          sums[4 * KV + k] = sums[4 * KV + k] + xv * yv
        return tuple(sums)

      def run_acc(pos, end, sums):
        return lax.fori_loop(pos, end, row_acc, tuple(sums))

      def find_end(pos, vid):
        vidv = jnp.full((16,), vid, dtype=jnp.int32)

        def fstep(g, found):
          def scan():
            q0 = g * 16
            w = idbuf[pl.ds(base + q0, 16)]
            m = jnp.logical_and(w != vidv, q0 + lane >= pos)
            f = plsc.all_reduce_ffs(m)[0]
            return jnp.where(f < 16, q0 + f, jnp.int32(-1))

          return lax.cond(found < 0, scan, lambda: found)

        found = lax.fori_loop(pos >> 4, NG, fstep, jnp.int32(-1))
        return found

      def seg_step(j, scarry):
        (pos, cur_id, cnt, rp, inflight) = scarry[:5]
        sums = scarry[5:]
        end = find_end(pos, cur_id)
        sums = run_acc(pos, end, tuple(sums))
        cnt = cnt + (end - pos)
        rp, inflight = flush_seg(cur_id, cnt, rp, inflight, sums)
        nid = idbuf[pl.ds(base + end, 16)][0]
        return (end, nid, jnp.int32(0), rp, inflight) + (zero16,) * 20

      scarry = (jnp.int32(0),) + tuple(carry)
      scarry = lax.fori_loop(0, np_, seg_step, scarry)
      (start, cur_id, cnt, rp, inflight) = scarry[:5]
      sums = run_acc(start, jnp.int32(R), tuple(scarry[5:]))
      cnt = cnt + (R - start)
      return (cur_id, cnt, rp, inflight) + tuple(sums)

    def block_pair(b2, carry):
      for p_ in (0, 1):
        carry = block_body_p(p_, b2 * 2 + p_, carry)
      return carry

    carry0 = (first_id, jnp.int32(0), jnp.int32(0), jnp.int32(0)) + \
        (zero16,) * 20
    carry = lax.fori_loop(0, NB // 2, block_pair, carry0)

    cur_id, cnt = carry[0], carry[1]
    inflight = carry[3]
    sums = carry[4:]

    last_diff = cur_id != first_id
    _store_slot(rec_stage, 0, sums, cnt)
    slot = jnp.where(last_diff, 1, 0).astype(jnp.int32)
    pltpu.sync_copy(
        rec_stage, shared_rec.at[pl.ds((chunk * 2 + slot) * REC, REC)])
    hdr_stage[pl.ds(0, 16)] = jnp.full((16,), first_id, dtype=jnp.int32)
    hdr_stage[pl.ds(16, 16)] = jnp.full(
        (16,), jnp.where(last_diff, cur_id, -1), dtype=jnp.int32)
    pltpu.sync_copy(hdr_stage, shared_hdr.at[pl.ds(chunk * 32, 32)])

    for j in range(RING):
      @pl.when(j < inflight)
      def _():
        drain_one_flush()

    plsc.subcore_barrier()

    @pl.when(chunk == 0)
    def _merge():
      pltpu.sync_copy(shared_rec, recbuf)
      pltpu.sync_copy(shared_hdr, hdrbuf)

      def flush_acc(prev):
        cntv = accbuf[pl.ds(5 * HF, 16)]
        pv = 1.0 / cntv
        moms = [accbuf[pl.ds(r * HF + 16 * k, 16)]
                for r in range(5) for k in range(KV)]
        corr = _corr_vregs(moms, pv)
        for k in range(KV):
          fin_stage[pl.ds(16 * k, 16)] = corr[k]
        pltpu.sync_copy(fin_stage, out_hbm.at[pl.ds(prev * D + f0, HF)])

      def slot_step(t, prev):
        hid = hdrbuf[pl.ds(t * 16, 16)][0]
        rb = t * REC

        def live(prev):
          same = hid == prev

          def addacc():
            for k in range(6 * KV):
              accbuf[pl.ds(16 * k, 16)] = (
                  accbuf[pl.ds(16 * k, 16)]
                  + recbuf[pl.ds(rb + 16 * k, 16)])

          def newacc():
            @pl.when(prev >= 0)
            def _():
              flush_acc(prev)
            for k in range(6 * KV):
              accbuf[pl.ds(16 * k, 16)] = recbuf[pl.ds(rb + 16 * k, 16)]

          lax.cond(same, addacc, newacc)
          return hid

        return lax.cond(hid >= 0, live, lambda p_: p_, prev)

      prev = lax.fori_loop(0, NCHUNK * 2, slot_step, jnp.int32(-1))

      @pl.when(prev >= 0)
      def _():
        flush_acc(prev)

  return pl.kernel(
      body,
      out_type=jax.ShapeDtypeStruct((S * D,), jnp.float32),
      mesh=mesh,
      compiler_params=pltpu.CompilerParams(use_tc_tiling_on_sc=False,
                                           needs_layout_passes=False),
      scratch_types=[
          pltpu.VMEM((2, R, HF), jnp.float32),
          pltpu.VMEM((2, R, HF), jnp.float32),
          pltpu.VMEM((2 * IDP,), jnp.int32),
          pltpu.VMEM((RING * HF,), jnp.float32),
          pltpu.VMEM((REC,), jnp.float32),
          pltpu.VMEM((32,), jnp.int32),
          pltpu.VMEM((NCHUNK * 2 * REC,), jnp.float32),
          pltpu.VMEM((NCHUNK * 2 * 16,), jnp.int32),
          pltpu.VMEM((REC,), jnp.float32),
          pltpu.VMEM((HF,), jnp.float32),
          pltpu.VMEM_SHARED((NCHUNK * 2 * REC,), jnp.float32),
          pltpu.VMEM_SHARED((NCHUNK * 2 * 16,), jnp.int32),
          pltpu.SemaphoreType.DMA,
          pltpu.SemaphoreType.DMA,
      ],
  )(x, y, idx)


def kernel(input, target, batch_idx):
  x = input.reshape(N, D).astype(jnp.float32)
  y = target.reshape(N, D).astype(jnp.float32)
  idx = batch_idx.reshape(N).astype(jnp.int32)
  return _sc_moments(x, y, idx).reshape(S, D)
